# baseline (speedup 1.0000x reference)
import functools

import jax
import jax.numpy as jnp
from jax import lax
from jax.experimental import pallas as pl
from jax.experimental.pallas import tpu as pltpu
from jax.experimental.pallas import tpu_sc as plsc


def _proj_body(t_ref, w_ref, b_ref, o_ref):
    o_ref[...] = (
        lax.dot_general(
            t_ref[...],
            w_ref[...],
            dimension_numbers=(((0,), (0,)), ((), ())),
            preferred_element_type=jnp.float32,
        )
        + b_ref[...]
    )


def _project_table(tableT, W, b):
    E, V = tableT.shape
    D = W.shape[1]
    blk = 14336
    grid = (V + blk - 1) // blk
    return pl.pallas_call(
        _proj_body,
        grid=(grid,),
        in_specs=[
            pl.BlockSpec((E, blk), lambda i: (0, i)),
            pl.BlockSpec((E, D), lambda i: (0, 0)),
            pl.BlockSpec((1, D), lambda i: (0, 0)),
        ],
        out_specs=pl.BlockSpec((blk, D), lambda i: (i, 0)),
        out_shape=jax.ShapeDtypeStruct((V, D), jnp.float32),
    )(tableT, W, b.reshape(1, D))


def _gather_rows(P, idx_flat):
    (B,) = idx_flat.shape
    D = P.shape[1]
    mesh = plsc.VectorSubcoreMesh(core_axis_name="c", subcore_axis_name="s")
    nw = mesh.num_cores * mesh.num_subcores
    b_per_w = B // nw

    chunk = 128
    nbuf = 6
    nchunk = b_per_w // chunk

    @functools.partial(
        pl.kernel,
        out_type=jax.ShapeDtypeStruct((B, D), jnp.float32),
        mesh=mesh,
        scratch_types=(
            [pltpu.VMEM((b_per_w,), jnp.int32)]
            + [pltpu.VMEM((chunk, D), jnp.float32)] * nbuf
            + [pltpu.SemaphoreType.DMA] * (2 * nbuf)
        ),
    )
    def k(p_hbm, i_hbm, o_hbm, idx_v, *rest):
        bufs = rest[:nbuf]
        gsems = rest[nbuf : 2 * nbuf]
        wsems = rest[2 * nbuf :]
        wid = lax.axis_index("s") * mesh.num_cores + lax.axis_index("c")
        base = wid * b_per_w
        pltpu.sync_copy(i_hbm.at[pl.ds(base, b_per_w)], idx_v)

        def gather(j, p):
            return pltpu.async_copy(
                p_hbm.at[idx_v.at[pl.ds(j * chunk, chunk)]], bufs[p], gsems[p]
            )

        def writeback(j, p):
            return pltpu.async_copy(
                bufs[p], o_hbm.at[pl.ds(base + j * chunk, chunk)], wsems[p]
            )

        g_h = [None] * nbuf
        w_h = [None] * nbuf
        for j in range(min(nbuf, nchunk)):
            g_h[j] = gather(j, j)
        for j in range(nchunk):
            p = j % nbuf
            g_h[p].wait()
            w_h[p] = writeback(j, p)
            nxt = j + nbuf
            if nxt < nchunk:
                w_h[p].wait()
                g_h[p] = gather(nxt, p)
        for h in w_h:
            if h is not None:
                h.wait()

    return k(P, idx_flat)


def kernel(morganSMILES, table, W, b):
    Bt, L = morganSMILES.shape
    D = W.shape[1]
    idx_flat = morganSMILES.T.astype(jnp.int32).reshape(-1)
    P = _project_table(table.T, W, b)
    out = _gather_rows(P, idx_flat)
    return out.reshape(L, Bt, D).transpose(1, 0, 2)


# speedup vs baseline: 1.0173x; 1.0173x over previous
import functools

import jax
import jax.numpy as jnp
from jax import lax
from jax.experimental import pallas as pl
from jax.experimental.pallas import tpu as pltpu
from jax.experimental.pallas import tpu_sc as plsc


def _proj_body(t_ref, w_ref, b_ref, o_ref):
    o_ref[...] = (
        lax.dot_general(
            t_ref[...],
            w_ref[...],
            dimension_numbers=(((0,), (0,)), ((), ())),
            preferred_element_type=jnp.float32,
        )
        + b_ref[...]
    )


def _project_table(tableT, W, b):
    E, V = tableT.shape
    D = W.shape[1]
    blk = 12800
    grid = (V + blk - 1) // blk
    return pl.pallas_call(
        _proj_body,
        grid=(grid,),
        in_specs=[
            pl.BlockSpec((E, blk), lambda i: (0, i)),
            pl.BlockSpec((E, D), lambda i: (0, 0)),
            pl.BlockSpec((1, D), lambda i: (0, 0)),
        ],
        out_specs=pl.BlockSpec((blk, D), lambda i: (i, 0)),
        out_shape=jax.ShapeDtypeStruct((V, D), jnp.float32),
    )(tableT, W, b.reshape(1, D))


def _gather_rows(P, idx_flat):
    (B,) = idx_flat.shape
    D = P.shape[1]
    mesh = plsc.VectorSubcoreMesh(core_axis_name="c", subcore_axis_name="s")
    nw = mesh.num_cores * mesh.num_subcores
    b_per_w = B // nw

    chunk = 200
    nbuf = 4
    nchunk = b_per_w // chunk

    @functools.partial(
        pl.kernel,
        out_type=jax.ShapeDtypeStruct((B, D), jnp.float32),
        mesh=mesh,
        scratch_types=(
            [pltpu.VMEM((b_per_w,), jnp.int32)]
            + [pltpu.VMEM((chunk, D), jnp.float32)] * nbuf
            + [pltpu.SemaphoreType.DMA] * (2 * nbuf)
        ),
    )
    def k(p_hbm, i_hbm, o_hbm, idx_v, *rest):
        bufs = rest[:nbuf]
        gsems = rest[nbuf : 2 * nbuf]
        wsems = rest[2 * nbuf :]
        wid = lax.axis_index("s") * mesh.num_cores + lax.axis_index("c")
        base = wid * b_per_w
        pltpu.sync_copy(i_hbm.at[pl.ds(base, b_per_w)], idx_v)

        def gather(j, p):
            return pltpu.async_copy(
                p_hbm.at[idx_v.at[pl.ds(j * chunk, chunk)]], bufs[p], gsems[p]
            )

        def writeback(j, p):
            return pltpu.async_copy(
                bufs[p], o_hbm.at[pl.ds(base + j * chunk, chunk)], wsems[p]
            )

        g_h = [None] * nbuf
        w_h = [None] * nbuf
        for j in range(min(nbuf, nchunk)):
            g_h[j] = gather(j, j)
        for j in range(nchunk):
            p = j % nbuf
            g_h[p].wait()
            w_h[p] = writeback(j, p)
            nxt = j + nbuf
            if nxt < nchunk:
                w_h[p].wait()
                g_h[p] = gather(nxt, p)
        for h in w_h:
            if h is not None:
                h.wait()

    return k(P, idx_flat)


def kernel(morganSMILES, table, W, b):
    Bt, L = morganSMILES.shape
    D = W.shape[1]
    idx_flat = morganSMILES.T.astype(jnp.int32).reshape(-1)
    P = _project_table(table.T, W, b)
    out = _gather_rows(P, idx_flat)
    return out.reshape(L, Bt, D).transpose(1, 0, 2)


# speedup vs baseline: 1.0189x; 1.0016x over previous
import functools

import jax
import jax.numpy as jnp
from jax import lax
from jax.experimental import pallas as pl
from jax.experimental.pallas import tpu as pltpu
from jax.experimental.pallas import tpu_sc as plsc


def _proj_body(t_ref, w_ref, b_ref, o_ref):
    o_ref[...] = (
        lax.dot_general(
            t_ref[...],
            w_ref[...],
            dimension_numbers=(((0,), (0,)), ((), ())),
            preferred_element_type=jnp.float32,
        )
        + b_ref[...]
    )


def _project_table(tableT, W, b):
    E, V = tableT.shape
    D = W.shape[1]
    blk = 12800
    grid = (V + blk - 1) // blk
    return pl.pallas_call(
        _proj_body,
        grid=(grid,),
        in_specs=[
            pl.BlockSpec((E, blk), lambda i: (0, i)),
            pl.BlockSpec((E, D), lambda i: (0, 0)),
            pl.BlockSpec((1, D), lambda i: (0, 0)),
        ],
        out_specs=pl.BlockSpec((blk, D), lambda i: (i, 0)),
        out_shape=jax.ShapeDtypeStruct((V, D), jnp.float32),
    )(tableT, W, b.reshape(1, D))


def _gather_rows(P, idx_flat):
    (B,) = idx_flat.shape
    D = P.shape[1]
    mesh = plsc.VectorSubcoreMesh(core_axis_name="c", subcore_axis_name="s")
    nw = mesh.num_cores * mesh.num_subcores
    b_per_w = B // nw

    chunk = 256
    nbuf = 3
    nchunk = b_per_w // chunk

    @functools.partial(
        pl.kernel,
        out_type=jax.ShapeDtypeStruct((B, D), jnp.float32),
        mesh=mesh,
        scratch_types=(
            [pltpu.VMEM((b_per_w,), jnp.int32)]
            + [pltpu.VMEM((chunk, D), jnp.float32)] * nbuf
            + [pltpu.SemaphoreType.DMA] * (2 * nbuf)
        ),
    )
    def k(p_hbm, i_hbm, o_hbm, idx_v, *rest):
        bufs = rest[:nbuf]
        gsems = rest[nbuf : 2 * nbuf]
        wsems = rest[2 * nbuf :]
        wid = lax.axis_index("s") * mesh.num_cores + lax.axis_index("c")
        base = wid * b_per_w
        pltpu.sync_copy(i_hbm.at[pl.ds(base, b_per_w)], idx_v)

        def gather(j, p):
            return pltpu.async_copy(
                p_hbm.at[idx_v.at[pl.ds(j * chunk, chunk)]], bufs[p], gsems[p]
            )

        def writeback(j, p):
            return pltpu.async_copy(
                bufs[p], o_hbm.at[pl.ds(base + j * chunk, chunk)], wsems[p]
            )

        g_h = [None] * nbuf
        w_h = [None] * nbuf
        for j in range(min(nbuf, nchunk)):
            g_h[j] = gather(j, j)
        for j in range(nchunk):
            p = j % nbuf
            g_h[p].wait()
            w_h[p] = writeback(j, p)
            nxt = j + nbuf
            if nxt < nchunk:
                w_h[p].wait()
                g_h[p] = gather(nxt, p)
        for h in w_h:
            if h is not None:
                h.wait()

    return k(P, idx_flat)


def kernel(morganSMILES, table, W, b):
    Bt, L = morganSMILES.shape
    D = W.shape[1]
    idx_flat = morganSMILES.T.astype(jnp.int32).reshape(-1)
    P = _project_table(table.T, W, b)
    out = _gather_rows(P, idx_flat)
    return out.reshape(L, Bt, D).transpose(1, 0, 2)
